# baseline (speedup 1.0000x reference)
import jax
import jax.numpy as jnp
from jax.experimental import pallas as pl
from jax.experimental.pallas import tpu as pltpu


def _stage1_body(adj_a_ref, adj_b_ref, feat_ref, rx_ref, W_ref, b_ref, Wm_ref,
                 bg_ref, out_ref):
    i = pl.program_id(0)
    tm = adj_a_ref.shape[0]
    na = feat_ref.shape[0] // (2 * tm)

    @pl.when(i < na)
    def _compute():
        feat = feat_ref[...]
        t = jnp.concatenate(
            (
                jnp.dot(adj_a_ref[...], feat, preferred_element_type=jnp.float32),
                jnp.dot(adj_b_ref[...], feat, preferred_element_type=jnp.float32),
            ),
            axis=0,
        )
        gcn = jnp.maximum(
            jnp.dot(t, W_ref[...], preferred_element_type=jnp.float32)
            + b_ref[...],
            0.0,
        )
        h = feat_ref[pl.ds(i * 2 * tm, 2 * tm), :]
        gate = jax.nn.sigmoid(
            jnp.dot(h, Wm_ref[...], preferred_element_type=jnp.float32)
            + bg_ref[...]
        )
        out_ref[...] = gate * gcn + (1.0 - gate) * h

    @pl.when(i >= na)
    def _copy_rx():
        out_ref[...] = rx_ref[...]


def _stage1(adj, feat, rx, W, b, Wm, bg, tm):
    m, k = adj.shape
    mr = rx.shape[0]
    d = feat.shape[1]
    na = m // (2 * tm)
    nr = mr // (2 * tm)
    last_a, last_b = 2 * na - 2, 2 * na - 1
    return pl.pallas_call(
        _stage1_body,
        grid=(na + nr,),
        in_specs=[
            pl.BlockSpec((tm, k), lambda i: (jnp.minimum(2 * i, last_a), 0)),
            pl.BlockSpec((tm, k), lambda i: (jnp.minimum(2 * i + 1, last_b), 0)),
            pl.BlockSpec((m, d), lambda i: (0, 0)),
            pl.BlockSpec((2 * tm, d), lambda i: (jnp.maximum(i - na, 0), 0)),
            pl.BlockSpec((d, d), lambda i: (0, 0)),
            pl.BlockSpec((1, d), lambda i: (0, 0)),
            pl.BlockSpec((d, d), lambda i: (0, 0)),
            pl.BlockSpec((1, d), lambda i: (0, 0)),
        ],
        out_specs=pl.BlockSpec((2 * tm, d), lambda i: (i, 0)),
        out_shape=jax.ShapeDtypeStruct((m + mr, d), jnp.float32),
        compiler_params=pltpu.CompilerParams(
            dimension_semantics=("arbitrary",),
        ),
    )(adj, adj, feat, rx, W, b, Wm, bg)


def _stage2_body(*refs):
    adj_refs = refs[:-6]
    feat_ref, W_ref, b_ref, Wm_ref, bg_ref, out_ref = refs[-6:]
    i = pl.program_id(0)
    tm = adj_refs[0].shape[0]
    ns = len(adj_refs)
    feat = feat_ref[...]
    t = jnp.concatenate(
        [jnp.dot(a[...], feat, preferred_element_type=jnp.float32)
         for a in adj_refs],
        axis=0,
    )
    gcn = jnp.maximum(
        jnp.dot(t, W_ref[...], preferred_element_type=jnp.float32) + b_ref[...],
        0.0,
    )
    h = feat_ref[pl.ds(i * ns * tm, ns * tm), :]
    gate = jax.nn.sigmoid(
        jnp.dot(h, Wm_ref[...], preferred_element_type=jnp.float32) + bg_ref[...]
    )
    out_ref[...] = gate * gcn + (1.0 - gate) * h


def _adj_spec(tm, k, ns, s):
    return pl.BlockSpec((tm, k), lambda i: (ns * i + s, 0))


def _stage2(adj, feat, W, b, Wm, bg, tm, ns):
    m, k = adj.shape
    d = feat.shape[1]
    return pl.pallas_call(
        _stage2_body,
        grid=(m // (ns * tm),),
        in_specs=[_adj_spec(tm, k, ns, s) for s in range(ns)] + [
            pl.BlockSpec((k, d), lambda i: (0, 0)),
            pl.BlockSpec((d, d), lambda i: (0, 0)),
            pl.BlockSpec((1, d), lambda i: (0, 0)),
            pl.BlockSpec((d, d), lambda i: (0, 0)),
            pl.BlockSpec((1, d), lambda i: (0, 0)),
        ],
        out_specs=pl.BlockSpec((ns * tm, d), lambda i: (i, 0)),
        out_shape=jax.ShapeDtypeStruct((m, d), jnp.float32),
        compiler_params=pltpu.CompilerParams(
            dimension_semantics=("arbitrary",),
        ),
    )(*([adj] * ns), feat, W, b, Wm, bg)


def kernel(e_x, r_x, prim_adj, rela_adj, W1, b1, Wm, bg, W2, b2):
    b1r = b1.reshape(1, -1)
    b2r = b2.reshape(1, -1)
    bgr = bg.reshape(1, -1)
    feat2 = _stage1(prim_adj, e_x, r_x, W1, b1r, Wm, bgr, tm=200)
    x2 = _stage2(rela_adj, feat2, W2, b2r, Wm, bgr, tm=120, ns=4)
    return x2


# speedup vs baseline: 1.0091x; 1.0091x over previous
import jax
import jax.numpy as jnp
from jax.experimental import pallas as pl
from jax.experimental.pallas import tpu as pltpu


def _stage1_body(adj_a_ref, adj_b_ref, feat_ref, rx_ref, W_ref, b_ref, Wm_ref,
                 bg_ref, out_ref):
    i = pl.program_id(0)
    tm = adj_a_ref.shape[0]
    na = feat_ref.shape[0] // (2 * tm)

    @pl.when(i < na)
    def _compute():
        feat = feat_ref[...]
        t = jnp.concatenate(
            (
                jnp.dot(adj_a_ref[...], feat, preferred_element_type=jnp.float32),
                jnp.dot(adj_b_ref[...], feat, preferred_element_type=jnp.float32),
            ),
            axis=0,
        )
        gcn = jnp.maximum(
            jnp.dot(t, W_ref[...], preferred_element_type=jnp.float32)
            + b_ref[...],
            0.0,
        )
        h = feat_ref[pl.ds(i * 2 * tm, 2 * tm), :]
        gate = jax.nn.sigmoid(
            jnp.dot(h, Wm_ref[...], preferred_element_type=jnp.float32)
            + bg_ref[...]
        )
        out_ref[...] = gate * gcn + (1.0 - gate) * h

    @pl.when(i >= na)
    def _copy_rx():
        out_ref[...] = rx_ref[...]


def _stage1(adj, feat, rx, W, b, Wm, bg, tm):
    m, k = adj.shape
    mr = rx.shape[0]
    d = feat.shape[1]
    na = m // (2 * tm)
    nr = mr // (2 * tm)
    last_a, last_b = 2 * na - 2, 2 * na - 1
    return pl.pallas_call(
        _stage1_body,
        grid=(na + nr,),
        in_specs=[
            pl.BlockSpec((tm, k), lambda i: (jnp.minimum(2 * i, last_a), 0)),
            pl.BlockSpec((tm, k), lambda i: (jnp.minimum(2 * i + 1, last_b), 0)),
            pl.BlockSpec((m, d), lambda i: (0, 0)),
            pl.BlockSpec((2 * tm, d), lambda i: (jnp.maximum(i - na, 0), 0)),
            pl.BlockSpec((d, d), lambda i: (0, 0)),
            pl.BlockSpec((1, d), lambda i: (0, 0)),
            pl.BlockSpec((d, d), lambda i: (0, 0)),
            pl.BlockSpec((1, d), lambda i: (0, 0)),
        ],
        out_specs=pl.BlockSpec((2 * tm, d), lambda i: (i, 0)),
        out_shape=jax.ShapeDtypeStruct((m + mr, d), jnp.float32),
        compiler_params=pltpu.CompilerParams(
            dimension_semantics=("arbitrary",),
        ),
    )(adj, adj, feat, rx, W, b, Wm, bg)


def _stage2_body(*refs):
    adj_refs = refs[:-6]
    feat_ref, W_ref, b_ref, Wm_ref, bg_ref, out_ref = refs[-6:]
    i = pl.program_id(0)
    tm = adj_refs[0].shape[0]
    ns = len(adj_refs)
    feat = feat_ref[...]
    t = jnp.concatenate(
        [jnp.dot(a[...], feat, preferred_element_type=jnp.float32)
         for a in adj_refs],
        axis=0,
    )
    gcn = jnp.maximum(
        jnp.dot(t, W_ref[...], preferred_element_type=jnp.float32) + b_ref[...],
        0.0,
    )
    h = feat_ref[pl.ds(i * ns * tm, ns * tm), :]
    gate = jax.nn.sigmoid(
        jnp.dot(h, Wm_ref[...], preferred_element_type=jnp.float32) + bg_ref[...]
    )
    out_ref[...] = gate * gcn + (1.0 - gate) * h


def _adj_spec(tm, k, ns, s):
    return pl.BlockSpec((tm, k), lambda i: (ns * i + s, 0))


def _stage2(adj, feat, W, b, Wm, bg, tm, ns):
    m, k = adj.shape
    d = feat.shape[1]
    return pl.pallas_call(
        _stage2_body,
        grid=(m // (ns * tm),),
        in_specs=[_adj_spec(tm, k, ns, s) for s in range(ns)] + [
            pl.BlockSpec((k, d), lambda i: (0, 0)),
            pl.BlockSpec((d, d), lambda i: (0, 0)),
            pl.BlockSpec((1, d), lambda i: (0, 0)),
            pl.BlockSpec((d, d), lambda i: (0, 0)),
            pl.BlockSpec((1, d), lambda i: (0, 0)),
        ],
        out_specs=pl.BlockSpec((ns * tm, d), lambda i: (i, 0)),
        out_shape=jax.ShapeDtypeStruct((m, d), jnp.float32),
        compiler_params=pltpu.CompilerParams(
            dimension_semantics=("arbitrary",),
        ),
    )(*([adj] * ns), feat, W, b, Wm, bg)


def kernel(e_x, r_x, prim_adj, rela_adj, W1, b1, Wm, bg, W2, b2):
    b1r = b1.reshape(1, -1)
    b2r = b2.reshape(1, -1)
    bgr = bg.reshape(1, -1)
    feat2 = _stage1(prim_adj, e_x, r_x, W1, b1r, Wm, bgr, tm=200)
    x2 = _stage2(rela_adj, feat2, W2, b2r, Wm, bgr, tm=200, ns=2)
    return x2
